# baseline (speedup 1.0000x reference)
import functools

import numpy as np
import jax
import jax.numpy as jnp
from jax import lax
from jax.experimental import pallas as pl
from jax.experimental.pallas import tpu as pltpu

EPS = 1e-5
VMEM_LIMIT = int(64 * 1024 * 1024 * 0.75)


def _gram_kernel(x_ref, out_ref):
    tn, lin, h = x_ref.shape
    g = jnp.zeros((lin, lin), jnp.float32)
    for n in range(tn):
        xn = x_ref[n]
        g = g + lax.dot_general(xn, xn, (((1,), (1,)), ((), ())),
                                preferred_element_type=jnp.float32)
    s = jnp.sum(x_ref[...], axis=(0, 2))[None, :]
    out_ref[...] = jnp.concatenate(
        [g, jnp.broadcast_to(s, (8, lin))], axis=0)


def _epilogue_kernel(parts_ref, w_ref, g_ref, b_ref, wa_ref, sh_ref,
                     *, wout, cout, count):
    nts = parts_ref.shape[0]
    lin = w_ref.shape[1]
    acc = parts_ref[0]
    for t in range(1, nts):
        acc = acc + parts_ref[t]
    gram = acc[:lin, :]
    rsum = acc[lin:lin + 1, :]
    w = w_ref[...]
    wg = jnp.dot(w, gram, preferred_element_type=jnp.float32)
    ssq = jnp.sum(w * wg, axis=1, keepdims=True)
    ssm = jnp.sum(w * rsum, axis=1, keepdims=True)
    s1 = jnp.zeros((cout, 1), jnp.float32)
    s2 = jnp.zeros((cout, 1), jnp.float32)
    for wo in range(wout):
        s1 = s1 + ssm[wo * cout:(wo + 1) * cout]
        s2 = s2 + ssq[wo * cout:(wo + 1) * cout]
    mean = s1 * (1.0 / count)
    var = jnp.maximum(s2 * (1.0 / count) - mean * mean, 0.0)
    scale = g_ref[:, :1] * lax.rsqrt(var + EPS)
    shift = b_ref[:, :1] - mean * scale
    scale_l = jnp.concatenate([scale] * wout, axis=0)
    shift_l = jnp.concatenate([shift] * wout, axis=0)
    wa_ref[...] = w * scale_l
    sh_ref[...] = jnp.broadcast_to(shift_l, sh_ref.shape)


def _apply_kernel(x_ref, w_ref, b_ref, y_ref):
    tn = x_ref.shape[0]
    w = w_ref[...]
    b = b_ref[:, :1]
    for n in range(tn):
        y = jnp.dot(w, x_ref[n], preferred_element_type=jnp.float32)
        y_ref[n] = (y + b).astype(y_ref.dtype)


def _fold_weight_t(weight, W, Wout):
    Cin, Cout = weight.shape[0], weight.shape[1]
    P = np.zeros((4, W, Wout), np.float32)
    for k in range(4):
        for wi in range(W):
            wo = 2 * wi - 1 + k
            if 0 <= wo < Wout:
                P[k, wi, wo] = 1.0
    wt = weight[:, :, 0, :].astype(jnp.float32)
    w_t = jnp.einsum("kwv,cdk->vdwc", jnp.asarray(P), wt)
    return w_t.reshape(Wout * Cout, W * Cin)


def _tiles(n, target):
    t = max(d for d in range(1, min(target, n) + 1) if n % d == 0)
    return t, n // t


@jax.jit
def kernel(x_nhwc, weight, gamma, beta):
    N, H, W, Cin = x_nhwc.shape
    Cout = weight.shape[1]
    Wout = 2 * W
    L_in, L_out = W * Cin, Wout * Cout
    dtype = x_nhwc.dtype

    w_t = _fold_weight_t(weight, W, Wout)
    xp = x_nhwc.transpose(0, 2, 3, 1).reshape(N, L_in, H)

    cparams = pltpu.CompilerParams(
        dimension_semantics=("parallel",), vmem_limit_bytes=VMEM_LIMIT)

    TN_S, nts = _tiles(N, 128)
    parts = pl.pallas_call(
        _gram_kernel,
        out_shape=jax.ShapeDtypeStruct((nts, L_in + 8, L_in), jnp.float32),
        grid=(nts,),
        in_specs=[pl.BlockSpec((TN_S, L_in, H), lambda i: (i, 0, 0))],
        out_specs=pl.BlockSpec((None, L_in + 8, L_in), lambda i: (i, 0, 0)),
        compiler_params=cparams,
    )(xp)

    gb = jnp.broadcast_to(gamma.astype(jnp.float32)[:, None], (Cout, 128))
    bb = jnp.broadcast_to(beta.astype(jnp.float32)[:, None], (Cout, 128))
    w_apply, shift_col = pl.pallas_call(
        functools.partial(_epilogue_kernel, wout=Wout, cout=Cout,
                          count=float(N * H * Wout)),
        out_shape=(jax.ShapeDtypeStruct((L_out, L_in), jnp.float32),
                   jax.ShapeDtypeStruct((L_out, 128), jnp.float32)),
        grid=(1,),
        in_specs=[pl.BlockSpec((nts, L_in + 8, L_in), lambda i: (0, 0, 0)),
                  pl.BlockSpec((L_out, L_in), lambda i: (0, 0)),
                  pl.BlockSpec((Cout, 128), lambda i: (0, 0)),
                  pl.BlockSpec((Cout, 128), lambda i: (0, 0))],
        out_specs=(pl.BlockSpec((L_out, L_in), lambda i: (0, 0)),
                   pl.BlockSpec((L_out, 128), lambda i: (0, 0))),
        compiler_params=pltpu.CompilerParams(vmem_limit_bytes=VMEM_LIMIT),
    )(parts, w_t, gb, bb)
    w_apply = w_apply.astype(dtype)

    TN_A, nta = _tiles(N, 32)
    yp = pl.pallas_call(
        _apply_kernel,
        out_shape=jax.ShapeDtypeStruct((N, L_out, H), dtype),
        grid=(nta,),
        in_specs=[pl.BlockSpec((TN_A, L_in, H), lambda i: (i, 0, 0)),
                  pl.BlockSpec((L_out, L_in), lambda i: (0, 0)),
                  pl.BlockSpec((L_out, 128), lambda i: (0, 0))],
        out_specs=pl.BlockSpec((TN_A, L_out, H), lambda i: (i, 0, 0)),
        compiler_params=cparams,
    )(xp, w_apply, shift_col)
    return yp.reshape(N, Wout, Cout, H).transpose(0, 3, 1, 2)


# speedup vs baseline: 1.0198x; 1.0198x over previous
import functools

import numpy as np
import jax
import jax.numpy as jnp
from jax import lax
from jax.experimental import pallas as pl
from jax.experimental.pallas import tpu as pltpu

EPS = 1e-5
VMEM_LIMIT = int(64 * 1024 * 1024 * 0.75)


def _gram_kernel(x_ref, out_ref):
    tn, lin, h = x_ref.shape
    g = jnp.zeros((lin, lin), jnp.float32)
    for n in range(tn):
        xn = x_ref[n]
        g = g + lax.dot_general(xn, xn, (((1,), (1,)), ((), ())),
                                preferred_element_type=jnp.float32)
    s = jnp.sum(x_ref[...], axis=(0, 2))[None, :]
    out_ref[...] = jnp.concatenate(
        [g, jnp.broadcast_to(s, (8, lin))], axis=0)


def _epilogue_kernel(parts_ref, w_ref, g_ref, b_ref, wa_ref, sh_ref,
                     *, wout, cout, count):
    nts = parts_ref.shape[0]
    lin = w_ref.shape[1]
    acc = parts_ref[0]
    for t in range(1, nts):
        acc = acc + parts_ref[t]
    gram = acc[:lin, :]
    rsum = acc[lin:lin + 1, :]
    w = w_ref[...]
    wg = jnp.dot(w, gram, preferred_element_type=jnp.float32)
    ssq = jnp.sum(w * wg, axis=1, keepdims=True)
    ssm = jnp.sum(w * rsum, axis=1, keepdims=True)
    s1 = jnp.zeros((cout, 1), jnp.float32)
    s2 = jnp.zeros((cout, 1), jnp.float32)
    for wo in range(wout):
        s1 = s1 + ssm[wo * cout:(wo + 1) * cout]
        s2 = s2 + ssq[wo * cout:(wo + 1) * cout]
    mean = s1 * (1.0 / count)
    var = jnp.maximum(s2 * (1.0 / count) - mean * mean, 0.0)
    scale = g_ref[:, :1] * lax.rsqrt(var + EPS)
    shift = b_ref[:, :1] - mean * scale
    scale_l = jnp.concatenate([scale] * wout, axis=0)
    shift_l = jnp.concatenate([shift] * wout, axis=0)
    wa_ref[...] = w * scale_l
    sh_ref[...] = jnp.broadcast_to(shift_l, sh_ref.shape)


def _apply_kernel(x_ref, w_ref, b_ref, y_ref):
    tn = x_ref.shape[0]
    w = w_ref[...]
    b = b_ref[:, :1]
    for n in range(tn):
        y = jnp.dot(w, x_ref[n], preferred_element_type=jnp.float32)
        y_ref[n] = (y + b).astype(y_ref.dtype)


def _fold_weight_t(weight, W, Wout):
    Cin, Cout = weight.shape[0], weight.shape[1]
    P = np.zeros((4, W, Wout), np.float32)
    for k in range(4):
        for wi in range(W):
            wo = 2 * wi - 1 + k
            if 0 <= wo < Wout:
                P[k, wi, wo] = 1.0
    wt = weight[:, :, 0, :].astype(jnp.float32)
    w_t = jnp.einsum("kwv,cdk->vdwc", jnp.asarray(P), wt)
    return w_t.reshape(Wout * Cout, W * Cin)


def _tiles(n, target):
    t = max(d for d in range(1, min(target, n) + 1) if n % d == 0)
    return t, n // t


@jax.jit
def kernel(x_nhwc, weight, gamma, beta):
    N, H, W, Cin = x_nhwc.shape
    Cout = weight.shape[1]
    Wout = 2 * W
    L_in, L_out = W * Cin, Wout * Cout
    dtype = x_nhwc.dtype

    w_t = _fold_weight_t(weight, W, Wout)
    xp = x_nhwc.transpose(0, 2, 3, 1).reshape(N, L_in, H)

    cparams = pltpu.CompilerParams(
        dimension_semantics=("parallel",), vmem_limit_bytes=VMEM_LIMIT)

    TN_S, nts = _tiles(N, 64)
    parts = pl.pallas_call(
        _gram_kernel,
        out_shape=jax.ShapeDtypeStruct((nts, L_in + 8, L_in), jnp.float32),
        grid=(nts,),
        in_specs=[pl.BlockSpec((TN_S, L_in, H), lambda i: (i, 0, 0))],
        out_specs=pl.BlockSpec((None, L_in + 8, L_in), lambda i: (i, 0, 0)),
        compiler_params=cparams,
    )(xp)

    gb = jnp.broadcast_to(gamma.astype(jnp.float32)[:, None], (Cout, 128))
    bb = jnp.broadcast_to(beta.astype(jnp.float32)[:, None], (Cout, 128))
    w_apply, shift_col = pl.pallas_call(
        functools.partial(_epilogue_kernel, wout=Wout, cout=Cout,
                          count=float(N * H * Wout)),
        out_shape=(jax.ShapeDtypeStruct((L_out, L_in), jnp.float32),
                   jax.ShapeDtypeStruct((L_out, 128), jnp.float32)),
        grid=(1,),
        in_specs=[pl.BlockSpec((nts, L_in + 8, L_in), lambda i: (0, 0, 0)),
                  pl.BlockSpec((L_out, L_in), lambda i: (0, 0)),
                  pl.BlockSpec((Cout, 128), lambda i: (0, 0)),
                  pl.BlockSpec((Cout, 128), lambda i: (0, 0))],
        out_specs=(pl.BlockSpec((L_out, L_in), lambda i: (0, 0)),
                   pl.BlockSpec((L_out, 128), lambda i: (0, 0))),
        compiler_params=pltpu.CompilerParams(vmem_limit_bytes=VMEM_LIMIT),
    )(parts, w_t, gb, bb)
    w_apply = w_apply.astype(dtype)

    TN_A, nta = _tiles(N, 32)
    yp = pl.pallas_call(
        _apply_kernel,
        out_shape=jax.ShapeDtypeStruct((N, L_out, H), dtype),
        grid=(nta,),
        in_specs=[pl.BlockSpec((TN_A, L_in, H), lambda i: (i, 0, 0)),
                  pl.BlockSpec((L_out, L_in), lambda i: (0, 0)),
                  pl.BlockSpec((L_out, 128), lambda i: (0, 0))],
        out_specs=pl.BlockSpec((TN_A, L_out, H), lambda i: (i, 0, 0)),
        compiler_params=cparams,
    )(xp, w_apply, shift_col)
    return yp.reshape(N, Wout, Cout, H).transpose(0, 3, 1, 2)


# speedup vs baseline: 1.0424x; 1.0221x over previous
import functools

import numpy as np
import jax
import jax.numpy as jnp
from jax import lax
from jax.experimental import pallas as pl
from jax.experimental.pallas import tpu as pltpu

EPS = 1e-5
VMEM_LIMIT = int(64 * 1024 * 1024 * 0.75)


def _gram_kernel(x_ref, out_ref):
    tn, lin, h = x_ref.shape
    g = jnp.zeros((lin, lin), jnp.float32)
    for n in range(tn):
        xn = x_ref[n]
        g = g + lax.dot_general(xn, xn, (((1,), (1,)), ((), ())),
                                preferred_element_type=jnp.float32)
    s = jnp.sum(x_ref[...], axis=(0, 2))[None, :]
    out_ref[...] = jnp.concatenate(
        [g, jnp.broadcast_to(s, (8, lin))], axis=0)


def _epilogue_kernel(parts_ref, wk_ref, g_ref, b_ref, wa_ref, sh_ref,
                     wt_ref, *, wout, cout, cin, w_in, count):
    nts = parts_ref.shape[0]
    lin = wt_ref.shape[1]
    wt_ref[...] = jnp.zeros(wt_ref.shape, jnp.float32)
    for k in range(4):
        blk = wk_ref[k * cout:(k + 1) * cout, :]
        for wi in range(w_in):
            wo = 2 * wi - 1 + k
            if 0 <= wo < wout:
                wt_ref[wo * cout:(wo + 1) * cout,
                       wi * cin:(wi + 1) * cin] = blk
    acc = parts_ref[0]
    for t in range(1, nts):
        acc = acc + parts_ref[t]
    gram = acc[:lin, :]
    rsum = acc[lin:lin + 1, :]
    w = wt_ref[...]
    wg = jnp.dot(w, gram, preferred_element_type=jnp.float32)
    ssq = jnp.sum(w * wg, axis=1, keepdims=True)
    ssm = jnp.sum(w * rsum, axis=1, keepdims=True)
    s1 = jnp.zeros((cout, 1), jnp.float32)
    s2 = jnp.zeros((cout, 1), jnp.float32)
    for wo in range(wout):
        s1 = s1 + ssm[wo * cout:(wo + 1) * cout]
        s2 = s2 + ssq[wo * cout:(wo + 1) * cout]
    mean = s1 * (1.0 / count)
    var = jnp.maximum(s2 * (1.0 / count) - mean * mean, 0.0)
    scale = g_ref[:, :1] * lax.rsqrt(var + EPS)
    shift = b_ref[:, :1] - mean * scale
    scale_l = jnp.concatenate([scale] * wout, axis=0)
    shift_l = jnp.concatenate([shift] * wout, axis=0)
    wa_ref[...] = w * scale_l
    sh_ref[...] = jnp.broadcast_to(shift_l, sh_ref.shape)


def _apply_kernel(x_ref, w_ref, b_ref, y_ref):
    tn = x_ref.shape[0]
    w = w_ref[...]
    b = b_ref[:, :1]
    for n in range(tn):
        y = jnp.dot(w, x_ref[n], preferred_element_type=jnp.float32)
        y_ref[n] = (y + b).astype(y_ref.dtype)


def _tiles(n, target):
    t = max(d for d in range(1, min(target, n) + 1) if n % d == 0)
    return t, n // t


@jax.jit
def kernel(x_nhwc, weight, gamma, beta):
    N, H, W, Cin = x_nhwc.shape
    Cout = weight.shape[1]
    Wout = 2 * W
    L_in, L_out = W * Cin, Wout * Cout
    dtype = x_nhwc.dtype

    wk = weight[:, :, 0, :].astype(jnp.float32).transpose(2, 1, 0)
    wk = wk.reshape(4 * Cout, Cin)
    xp = x_nhwc.transpose(0, 2, 3, 1).reshape(N, L_in, H)

    cparams = pltpu.CompilerParams(
        dimension_semantics=("parallel",), vmem_limit_bytes=VMEM_LIMIT)

    TN_S, nts = _tiles(N, 64)
    parts = pl.pallas_call(
        _gram_kernel,
        out_shape=jax.ShapeDtypeStruct((nts, L_in + 8, L_in), jnp.float32),
        grid=(nts,),
        in_specs=[pl.BlockSpec((TN_S, L_in, H), lambda i: (i, 0, 0))],
        out_specs=pl.BlockSpec((None, L_in + 8, L_in), lambda i: (i, 0, 0)),
        compiler_params=cparams,
    )(xp)

    gb = jnp.broadcast_to(gamma.astype(jnp.float32)[:, None], (Cout, 128))
    bb = jnp.broadcast_to(beta.astype(jnp.float32)[:, None], (Cout, 128))
    w_apply, shift_col = pl.pallas_call(
        functools.partial(_epilogue_kernel, wout=Wout, cout=Cout, cin=Cin,
                          w_in=W, count=float(N * H * Wout)),
        out_shape=(jax.ShapeDtypeStruct((L_out, L_in), jnp.float32),
                   jax.ShapeDtypeStruct((L_out, 128), jnp.float32)),
        grid=(1,),
        in_specs=[pl.BlockSpec((nts, L_in + 8, L_in), lambda i: (0, 0, 0)),
                  pl.BlockSpec((4 * Cout, Cin), lambda i: (0, 0)),
                  pl.BlockSpec((Cout, 128), lambda i: (0, 0)),
                  pl.BlockSpec((Cout, 128), lambda i: (0, 0))],
        out_specs=(pl.BlockSpec((L_out, L_in), lambda i: (0, 0)),
                   pl.BlockSpec((L_out, 128), lambda i: (0, 0))),
        scratch_shapes=[pltpu.VMEM((L_out, L_in), jnp.float32)],
        compiler_params=pltpu.CompilerParams(vmem_limit_bytes=VMEM_LIMIT),
    )(parts, wk, gb, bb)
    w_apply = w_apply.astype(dtype)

    TN_A, nta = _tiles(N, 32)
    yp = pl.pallas_call(
        _apply_kernel,
        out_shape=jax.ShapeDtypeStruct((N, L_out, H), dtype),
        grid=(nta,),
        in_specs=[pl.BlockSpec((TN_A, L_in, H), lambda i: (i, 0, 0)),
                  pl.BlockSpec((L_out, L_in), lambda i: (0, 0)),
                  pl.BlockSpec((L_out, 128), lambda i: (0, 0))],
        out_specs=pl.BlockSpec((TN_A, L_out, H), lambda i: (i, 0, 0)),
        compiler_params=cparams,
    )(xp, w_apply, shift_col)
    return yp.reshape(N, Wout, Cout, H).transpose(0, 3, 1, 2)


# speedup vs baseline: 1.0436x; 1.0012x over previous
import functools

import jax
import jax.numpy as jnp
from jax import lax
from jax.experimental import pallas as pl
from jax.experimental.pallas import tpu as pltpu

EPS = 1e-5
VMEM_LIMIT = int(64 * 1024 * 1024 * 0.75)


def _gram_kernel(x_ref, out_ref):
    tn, lin, h = x_ref.shape
    g = jnp.zeros((lin, lin), jnp.float32)
    for n in range(tn):
        xn = x_ref[n]
        g = g + lax.dot_general(xn, xn, (((1,), (1,)), ((), ())),
                                preferred_element_type=jnp.float32)
    s = jnp.sum(x_ref[...], axis=(0, 2))[None, :]
    out_ref[...] = jnp.concatenate(
        [g, jnp.broadcast_to(s, (8, lin))], axis=0)


def _epilogue_kernel(parts_ref, wk_ref, g_ref, b_ref, wa_ref, sh_ref,
                     wt_ref, *, wout, cout, cin, w_in, count):
    nts = parts_ref.shape[0]
    lin = wt_ref.shape[1]
    wt_ref[...] = jnp.zeros(wt_ref.shape, jnp.float32)
    for k in range(4):
        blk = wk_ref[k * cout:(k + 1) * cout, :]
        for wi in range(w_in):
            wo = 2 * wi - 1 + k
            if 0 <= wo < wout:
                wt_ref[wo * cout:(wo + 1) * cout,
                       wi * cin:(wi + 1) * cin] = blk
    acc = parts_ref[0]
    for t in range(1, nts):
        acc = acc + parts_ref[t]
    gram = acc[:lin, :]
    rsum = acc[lin:lin + 1, :]
    w = wt_ref[...]
    wg = jnp.dot(w, gram, preferred_element_type=jnp.float32)
    ssq = jnp.sum(w * wg, axis=1, keepdims=True)
    ssm = jnp.sum(w * rsum, axis=1, keepdims=True)
    s1 = jnp.zeros((cout, 1), jnp.float32)
    s2 = jnp.zeros((cout, 1), jnp.float32)
    for wo in range(wout):
        s1 = s1 + ssm[wo * cout:(wo + 1) * cout]
        s2 = s2 + ssq[wo * cout:(wo + 1) * cout]
    mean = s1 * (1.0 / count)
    var = jnp.maximum(s2 * (1.0 / count) - mean * mean, 0.0)
    scale = g_ref[:, :1] * lax.rsqrt(var + EPS)
    shift = b_ref[:, :1] - mean * scale
    scale_l = jnp.concatenate([scale] * wout, axis=0)
    shift_l = jnp.concatenate([shift] * wout, axis=0)
    wa_ref[...] = w * scale_l
    sh_ref[...] = jnp.broadcast_to(shift_l, sh_ref.shape)


def _apply_kernel(x_ref, w_ref, b_ref, y_ref):
    tn = x_ref.shape[0]
    w = w_ref[...]
    b = b_ref[:, :1]
    for n in range(tn):
        y = jnp.dot(w, x_ref[n], preferred_element_type=jnp.float32)
        y_ref[n] = (y + b).astype(y_ref.dtype)


def _tiles(n, target):
    t = max(d for d in range(1, min(target, n) + 1) if n % d == 0)
    return t, n // t


@jax.jit
def kernel(x_nhwc, weight, gamma, beta):
    N, H, W, Cin = x_nhwc.shape
    Cout = weight.shape[1]
    Wout = 2 * W
    L_in, L_out = W * Cin, Wout * Cout
    dtype = x_nhwc.dtype

    wk = weight[:, :, 0, :].astype(jnp.float32).transpose(2, 1, 0)
    wk = wk.reshape(4 * Cout, Cin)
    xp = x_nhwc.transpose(0, 2, 3, 1).reshape(N, L_in, H)

    cparams = pltpu.CompilerParams(
        dimension_semantics=("parallel",), vmem_limit_bytes=VMEM_LIMIT)

    TN_S, nts = _tiles(N, 64)
    parts = pl.pallas_call(
        _gram_kernel,
        out_shape=jax.ShapeDtypeStruct((nts, L_in + 8, L_in), jnp.float32),
        grid=(nts,),
        in_specs=[pl.BlockSpec((TN_S, L_in, H), lambda i: (i, 0, 0))],
        out_specs=pl.BlockSpec((None, L_in + 8, L_in), lambda i: (i, 0, 0)),
        compiler_params=cparams,
    )(xp)

    gb = jnp.broadcast_to(gamma.astype(jnp.float32)[:, None], (Cout, 128))
    bb = jnp.broadcast_to(beta.astype(jnp.float32)[:, None], (Cout, 128))
    w_apply, shift_col = pl.pallas_call(
        functools.partial(_epilogue_kernel, wout=Wout, cout=Cout, cin=Cin,
                          w_in=W, count=float(N * H * Wout)),
        out_shape=(jax.ShapeDtypeStruct((L_out, L_in), jnp.float32),
                   jax.ShapeDtypeStruct((L_out, 128), jnp.float32)),
        grid=(1,),
        in_specs=[pl.BlockSpec((nts, L_in + 8, L_in), lambda i: (0, 0, 0)),
                  pl.BlockSpec((4 * Cout, Cin), lambda i: (0, 0)),
                  pl.BlockSpec((Cout, 128), lambda i: (0, 0)),
                  pl.BlockSpec((Cout, 128), lambda i: (0, 0))],
        out_specs=(pl.BlockSpec((L_out, L_in), lambda i: (0, 0)),
                   pl.BlockSpec((L_out, 128), lambda i: (0, 0))),
        scratch_shapes=[pltpu.VMEM((L_out, L_in), jnp.float32)],
        compiler_params=pltpu.CompilerParams(vmem_limit_bytes=VMEM_LIMIT),
    )(parts, wk, gb, bb)
    w_apply = w_apply.astype(dtype)

    TN_A, nta = _tiles(N, 32)
    yp = pl.pallas_call(
        _apply_kernel,
        out_shape=jax.ShapeDtypeStruct((N, L_out, H), dtype),
        grid=(nta,),
        in_specs=[pl.BlockSpec((TN_A, L_in, H), lambda i: (i, 0, 0)),
                  pl.BlockSpec((L_out, L_in), lambda i: (0, 0)),
                  pl.BlockSpec((L_out, 128), lambda i: (0, 0))],
        out_specs=pl.BlockSpec((TN_A, L_out, H), lambda i: (i, 0, 0)),
        compiler_params=cparams,
    )(xp, w_apply, shift_col)
    return yp.reshape(N, Wout, Cout, H).transpose(0, 3, 1, 2)
